# baseline (speedup 1.0000x reference)
import functools

import jax
import jax.numpy as jnp
from jax import lax
from jax.experimental import pallas as pl
from jax.experimental.pallas import tpu as pltpu
from jax.experimental.pallas import tpu_sc as plsc

N = 10000
E = 320000
D = 128

NC = 2
NS = 16
NW = NC * NS
CHUNK = 128
NCHUNK = 80
E_PAD = NW * NCHUNK * CHUNK
NBUF = 2
NIDX = 4
NPAD = 10112
ROWS_PER_SUBCORE = NPAD // NS


def _segment_sum_sc(h, ei):
    mesh = plsc.VectorSubcoreMesh(core_axis_name="c", subcore_axis_name="s")

    @functools.partial(
        pl.kernel,
        out_type=jax.ShapeDtypeStruct((NC, N, D), jnp.float32),
        mesh=mesh,
        scratch_types=[
            pltpu.VMEM((NIDX, 2, CHUNK), jnp.int32),
            pltpu.VMEM((NBUF, CHUNK, D), jnp.float32),
            pltpu.VMEM_SHARED((NPAD, D), jnp.float32),
            pltpu.SemaphoreType.DMA,
            pltpu.SemaphoreType.DMA,
            pltpu.SemaphoreType.DMA,
            pltpu.SemaphoreType.DMA,
            pltpu.SemaphoreType.DMA,
            pltpu.SemaphoreType.DMA,
        ],
    )
    def seg_kernel(h_hbm, ei_hbm, out_hbm, idx_v, rows_v, table,
                   gsem0, gsem1, isem0, isem1, isem2, isem3):
        gsems = [gsem0, gsem1]
        isems = [isem0, isem1, isem2, isem3]
        cid = lax.axis_index("c")
        sid = lax.axis_index("s")
        wid = sid * NC + cid

        def idx_start(c, q):
            pltpu.async_copy(ei_hbm.at[wid, c], idx_v.at[q], isems[q])

        def idx_wait(c, q):
            pltpu.make_async_copy(
                ei_hbm.at[wid, c], idx_v.at[q], isems[q]).wait()

        def gather_start(q, b):
            pltpu.async_copy(
                h_hbm.at[idx_v.at[q, 0]], rows_v.at[b], gsems[b])

        def gather_wait(q, b):
            pltpu.make_async_copy(
                h_hbm.at[idx_v.at[q, 0]], rows_v.at[b], gsems[b]).wait()

        @pl.loop(0, CHUNK)
        def _(r):
            @pl.loop(0, D, step=16)
            def _(c0):
                rows_v.at[0, r, pl.ds(c0, 16)][...] = jnp.zeros(
                    (16,), jnp.float32)

        row0 = sid * ROWS_PER_SUBCORE
        for k in range(ROWS_PER_SUBCORE // CHUNK):
            pltpu.sync_copy(rows_v.at[0],
                            table.at[pl.ds(row0 + k * CHUNK, CHUNK)])
        _rem = ROWS_PER_SUBCORE % CHUNK
        pltpu.sync_copy(
            rows_v.at[0, pl.ds(0, _rem)],
            table.at[pl.ds(row0 + ROWS_PER_SUBCORE - _rem, _rem)])
        plsc.subcore_barrier()

        for q in range(NIDX):
            idx_start(q, q)

        @pl.loop(0, NCHUNK, step=NIDX)
        def _(j):
            for k in range(NIDX):
                c = j + k
                idx_wait(c, k)
                pltpu.async_copy(
                    h_hbm.at[idx_v.at[k, 0]], rows_v.at[0], gsems[0]).wait()
                pltpu.sync_copy(rows_v.at[0], table.at[idx_v.at[k, 1]],
                                add=True)

                @pl.when(c + NIDX < NCHUNK)
                def _():
                    idx_start(c + NIDX, k)

        plsc.subcore_barrier()

        @pl.when(row0 + ROWS_PER_SUBCORE <= N)
        def _():
            pltpu.sync_copy(table.at[pl.ds(row0, ROWS_PER_SUBCORE)],
                            out_hbm.at[cid, pl.ds(row0, ROWS_PER_SUBCORE)])

        @pl.when(row0 + ROWS_PER_SUBCORE > N)
        def _():
            pltpu.sync_copy(table.at[pl.ds(row0, N % ROWS_PER_SUBCORE)],
                            out_hbm.at[cid, pl.ds(row0, N % ROWS_PER_SUBCORE)])

    return seg_kernel(h, ei)


_BLK = 1000


def _mlp_body(eps_ref, h_ref, agg_ref, w1_ref, b1_ref, w2_ref, b2_ref, o_ref):
    z = (1.0 + eps_ref[0]) * h_ref[...] + agg_ref[0] + agg_ref[1]
    t = jnp.maximum(
        jnp.dot(z, w1_ref[...], preferred_element_type=jnp.float32)
        + b1_ref[...], 0.0)
    o = jnp.maximum(
        jnp.dot(t, w2_ref[...], preferred_element_type=jnp.float32)
        + b2_ref[...], 0.0)
    o_ref[...] = o


def _gin_mlp_tc(h, agg, W1, b1, W2, b2, eps):
    grid = (N // _BLK,)
    return pl.pallas_call(
        _mlp_body,
        grid=grid,
        in_specs=[
            pl.BlockSpec(memory_space=pltpu.SMEM),
            pl.BlockSpec((_BLK, D), lambda i: (i, 0)),
            pl.BlockSpec((NC, _BLK, D), lambda i: (0, i, 0)),
            pl.BlockSpec((D, 2 * D), lambda i: (0, 0)),
            pl.BlockSpec((1, 2 * D), lambda i: (0, 0)),
            pl.BlockSpec((2 * D, D), lambda i: (0, 0)),
            pl.BlockSpec((1, D), lambda i: (0, 0)),
        ],
        out_specs=pl.BlockSpec((_BLK, D), lambda i: (i, 0)),
        out_shape=jax.ShapeDtypeStruct((N, D), jnp.float32),
    )(eps.reshape(1), h, agg, W1, b1.reshape(1, -1), W2, b2.reshape(1, -1))


def _final_body(h0_ref, h1_ref, h2_ref, h3_ref, w_ref, b_ref, o_ref):
    w = w_ref[...]
    o = jnp.dot(h0_ref[...], w[0 * D:1 * D], preferred_element_type=jnp.float32)
    o += jnp.dot(h1_ref[...], w[1 * D:2 * D], preferred_element_type=jnp.float32)
    o += jnp.dot(h2_ref[...], w[2 * D:3 * D], preferred_element_type=jnp.float32)
    o += jnp.dot(h3_ref[...], w[3 * D:4 * D], preferred_element_type=jnp.float32)
    o_ref[...] = o + b_ref[...]


def _final_linear_tc(h0, h1, h2, h3, lin_W, lin_b):
    grid = (N // _BLK,)
    row_spec = pl.BlockSpec((_BLK, D), lambda i: (i, 0))
    return pl.pallas_call(
        _final_body,
        grid=grid,
        in_specs=[
            row_spec, row_spec, row_spec, row_spec,
            pl.BlockSpec((4 * D, D), lambda i: (0, 0)),
            pl.BlockSpec((1, D), lambda i: (0, 0)),
        ],
        out_specs=row_spec,
        out_shape=jax.ShapeDtypeStruct((N, D), jnp.float32),
    )(h0, h1, h2, h3, lin_W, lin_b.reshape(1, -1))


def kernel(x, edge_index, W1_0, b1_0, W2_0, b2_0, eps_0,
           W1_1, b1_1, W2_1, b2_1, eps_1,
           W1_2, b1_2, W2_2, b2_2, eps_2, lin_W, lin_b):
    params = [
        (W1_0, b1_0, W2_0, b2_0, eps_0),
        (W1_1, b1_1, W2_1, b2_1, eps_1),
        (W1_2, b1_2, W2_2, b2_2, eps_2),
    ]
    pad = E_PAD - E
    src = jnp.concatenate(
        [edge_index[0], jnp.zeros((pad,), jnp.int32)]).reshape(
            NW, NCHUNK, CHUNK)
    pad_dst = N + (jnp.arange(pad, dtype=jnp.int32) % (NPAD - N))
    dst = jnp.concatenate([edge_index[1], pad_dst]).reshape(
        NW, NCHUNK, CHUNK)
    ei = jnp.stack([src, dst], axis=2)
    h = x
    h_list = [x]
    for (W1, b1, W2, b2, eps) in params:
        agg = _segment_sum_sc(h, ei)
        h = _gin_mlp_tc(h, agg, W1, b1, W2, b2, eps)
        h_list.append(h)
    return _final_linear_tc(h_list[0], h_list[1], h_list[2], h_list[3],
                            lin_W, lin_b)


# speedup vs baseline: 1.1316x; 1.1316x over previous
import functools

import jax
import jax.numpy as jnp
from jax import lax
from jax.experimental import pallas as pl
from jax.experimental.pallas import tpu as pltpu
from jax.experimental.pallas import tpu_sc as plsc

N = 10000
E = 320000
D = 128

NC = 2
NS = 16
NW = NC * NS
CHUNK = 128
NCHUNK = 80
E_PAD = NW * NCHUNK * CHUNK
NBUF = 2
NIDX = 4
NPAD = 10112
ROWS_PER_SUBCORE = NPAD // NS


def _segment_sum_sc(h, ei):
    mesh = plsc.VectorSubcoreMesh(core_axis_name="c", subcore_axis_name="s")

    @functools.partial(
        pl.kernel,
        out_type=jax.ShapeDtypeStruct((NC, N, D), jnp.float32),
        mesh=mesh,
        scratch_types=[
            pltpu.VMEM((2, CHUNK), jnp.int32),
            pltpu.VMEM((2, CHUNK), jnp.int32),
            pltpu.VMEM((2, CHUNK), jnp.int32),
            pltpu.VMEM((2, CHUNK), jnp.int32),
            pltpu.VMEM((CHUNK, D), jnp.float32),
            pltpu.VMEM((CHUNK, D), jnp.float32),
            pltpu.VMEM_SHARED((NPAD, D), jnp.float32),
            pltpu.SemaphoreType.DMA,
            pltpu.SemaphoreType.DMA,
            pltpu.SemaphoreType.DMA,
            pltpu.SemaphoreType.DMA,
            pltpu.SemaphoreType.DMA,
            pltpu.SemaphoreType.DMA,
        ],
    )
    def seg_kernel(h_hbm, ei_hbm, out_hbm, idx0, idx1, idx2, idx3,
                   rows0, rows1, table,
                   gsem0, gsem1, isem0, isem1, isem2, isem3):
        idxs = [idx0, idx1, idx2, idx3]
        rows = [rows0, rows1]
        gsems = [gsem0, gsem1]
        isems = [isem0, isem1, isem2, isem3]
        cid = lax.axis_index("c")
        sid = lax.axis_index("s")
        wid = sid * NC + cid

        def idx_start(c, q):
            pltpu.async_copy(ei_hbm.at[wid, c], idxs[q], isems[q])

        def idx_wait(c, q):
            pltpu.make_async_copy(
                ei_hbm.at[wid, c], idxs[q], isems[q]).wait()

        def gather_start(q, b):
            pltpu.async_copy(
                h_hbm.at[idxs[q].at[0]], rows[b], gsems[b])

        def gather_wait(q, b):
            pltpu.make_async_copy(
                h_hbm.at[idxs[q].at[0]], rows[b], gsems[b]).wait()

        @pl.loop(0, CHUNK)
        def _(r):
            @pl.loop(0, D, step=16)
            def _(c0):
                rows0.at[r, pl.ds(c0, 16)][...] = jnp.zeros(
                    (16,), jnp.float32)

        row0 = sid * ROWS_PER_SUBCORE
        for k in range(ROWS_PER_SUBCORE // CHUNK):
            pltpu.sync_copy(rows0,
                            table.at[pl.ds(row0 + k * CHUNK, CHUNK)])
        _rem = ROWS_PER_SUBCORE % CHUNK
        pltpu.sync_copy(
            rows0.at[pl.ds(0, _rem)],
            table.at[pl.ds(row0 + ROWS_PER_SUBCORE - _rem, _rem)])
        plsc.subcore_barrier()

        for q in range(NIDX):
            idx_start(q, q)
        for b in range(NBUF):
            idx_wait(b, b)
            gather_start(b, b)

        @pl.loop(0, NCHUNK, step=NIDX)
        def _(j):
            for k in range(NIDX):
                c = j + k
                b = k % NBUF
                gather_wait(k, b)
                pltpu.sync_copy(rows[b], table.at[idxs[k].at[1]],
                                add=True)

                @pl.when(c + NIDX < NCHUNK)
                def _():
                    idx_start(c + NIDX, k)

                @pl.when(c + NBUF < NCHUNK)
                def _():
                    idx_wait(c + NBUF, (k + NBUF) % NIDX)
                    gather_start((k + NBUF) % NIDX, b)

        plsc.subcore_barrier()

        @pl.when(row0 + ROWS_PER_SUBCORE <= N)
        def _():
            pltpu.sync_copy(table.at[pl.ds(row0, ROWS_PER_SUBCORE)],
                            out_hbm.at[cid, pl.ds(row0, ROWS_PER_SUBCORE)])

        @pl.when(row0 + ROWS_PER_SUBCORE > N)
        def _():
            pltpu.sync_copy(table.at[pl.ds(row0, N % ROWS_PER_SUBCORE)],
                            out_hbm.at[cid, pl.ds(row0, N % ROWS_PER_SUBCORE)])

    return seg_kernel(h, ei)


_BLK = 1000


def _mlp_body(eps_ref, h_ref, agg_ref, w1_ref, b1_ref, w2_ref, b2_ref, o_ref):
    z = (1.0 + eps_ref[0]) * h_ref[...] + agg_ref[0] + agg_ref[1]
    t = jnp.maximum(
        jnp.dot(z, w1_ref[...], preferred_element_type=jnp.float32)
        + b1_ref[...], 0.0)
    o = jnp.maximum(
        jnp.dot(t, w2_ref[...], preferred_element_type=jnp.float32)
        + b2_ref[...], 0.0)
    o_ref[...] = o


def _gin_mlp_tc(h, agg, W1, b1, W2, b2, eps):
    grid = (N // _BLK,)
    return pl.pallas_call(
        _mlp_body,
        grid=grid,
        in_specs=[
            pl.BlockSpec(memory_space=pltpu.SMEM),
            pl.BlockSpec((_BLK, D), lambda i: (i, 0)),
            pl.BlockSpec((NC, _BLK, D), lambda i: (0, i, 0)),
            pl.BlockSpec((D, 2 * D), lambda i: (0, 0)),
            pl.BlockSpec((1, 2 * D), lambda i: (0, 0)),
            pl.BlockSpec((2 * D, D), lambda i: (0, 0)),
            pl.BlockSpec((1, D), lambda i: (0, 0)),
        ],
        out_specs=pl.BlockSpec((_BLK, D), lambda i: (i, 0)),
        out_shape=jax.ShapeDtypeStruct((N, D), jnp.float32),
    )(eps.reshape(1), h, agg, W1, b1.reshape(1, -1), W2, b2.reshape(1, -1))


def _final_body(h0_ref, h1_ref, h2_ref, h3_ref, w_ref, b_ref, o_ref):
    w = w_ref[...]
    o = jnp.dot(h0_ref[...], w[0 * D:1 * D], preferred_element_type=jnp.float32)
    o += jnp.dot(h1_ref[...], w[1 * D:2 * D], preferred_element_type=jnp.float32)
    o += jnp.dot(h2_ref[...], w[2 * D:3 * D], preferred_element_type=jnp.float32)
    o += jnp.dot(h3_ref[...], w[3 * D:4 * D], preferred_element_type=jnp.float32)
    o_ref[...] = o + b_ref[...]


def _final_linear_tc(h0, h1, h2, h3, lin_W, lin_b):
    grid = (N // _BLK,)
    row_spec = pl.BlockSpec((_BLK, D), lambda i: (i, 0))
    return pl.pallas_call(
        _final_body,
        grid=grid,
        in_specs=[
            row_spec, row_spec, row_spec, row_spec,
            pl.BlockSpec((4 * D, D), lambda i: (0, 0)),
            pl.BlockSpec((1, D), lambda i: (0, 0)),
        ],
        out_specs=row_spec,
        out_shape=jax.ShapeDtypeStruct((N, D), jnp.float32),
    )(h0, h1, h2, h3, lin_W, lin_b.reshape(1, -1))


def kernel(x, edge_index, W1_0, b1_0, W2_0, b2_0, eps_0,
           W1_1, b1_1, W2_1, b2_1, eps_1,
           W1_2, b1_2, W2_2, b2_2, eps_2, lin_W, lin_b):
    params = [
        (W1_0, b1_0, W2_0, b2_0, eps_0),
        (W1_1, b1_1, W2_1, b2_1, eps_1),
        (W1_2, b1_2, W2_2, b2_2, eps_2),
    ]
    pad = E_PAD - E
    src = jnp.concatenate(
        [edge_index[0], jnp.zeros((pad,), jnp.int32)]).reshape(
            NW, NCHUNK, CHUNK)
    pad_dst = N + (jnp.arange(pad, dtype=jnp.int32) % (NPAD - N))
    dst = jnp.concatenate([edge_index[1], pad_dst]).reshape(
        NW, NCHUNK, CHUNK)
    ei = jnp.stack([src, dst], axis=2)
    h = x
    h_list = [x]
    for (W1, b1, W2, b2, eps) in params:
        agg = _segment_sum_sc(h, ei)
        h = _gin_mlp_tc(h, agg, W1, b1, W2, b2, eps)
        h_list.append(h)
    return _final_linear_tc(h_list[0], h_list[1], h_list[2], h_list[3],
                            lin_W, lin_b)
